# baseline (speedup 1.0000x reference)
import jax
import jax.numpy as jnp
from jax.experimental import pallas as pl
from jax.experimental.pallas import tpu as pltpu

_S = 3
_V = 25
_GF = 5
_GL = _GF * _V
_NB = 4


def _pack_lanes_const(a):
    _, TV = a.shape
    G = -(-TV // _GL)
    a = jnp.pad(a, ((0, 0), (0, G * _GL - TV)))
    a = a.reshape(1, G, _GL)
    a = jnp.pad(a, ((0, 0), (0, 0), (0, 128 - _GL)))
    return a.reshape(1, G * 128)


def _msg_kernel(x_ref, b_ref, w_ref, bias_ref, mask_ref,
                y_ref, s1_ref, s2_ref, xp_ref, agg_ref):
    NB, C, V, T = x_ref.shape
    L = xp_ref.shape[-1]
    TV = T * V
    xt = jnp.transpose(x_ref[...], (0, 1, 3, 2))
    X = xt.reshape(NB * C, TV)
    ngo = -(-TV // _GL)
    for g in range(ngo):
        w = min(_GL, TV - _GL * g)
        xp_ref[:, 128 * g:128 * g + w] = \
            X[:, _GL * g:_GL * g + w].astype(jnp.bfloat16)
        xp_ref[:, 128 * g + w:128 * (g + 1)] = \
            jnp.zeros((NB * C, 128 - w), jnp.bfloat16)
    for g in range(ngo, L // 128):
        xp_ref[:, 128 * g:128 * (g + 1)] = jnp.zeros((NB * C, 128),
                                                     jnp.bfloat16)
    Xp = xp_ref[...]
    for s in range(_S):
        for c in range(L // 256):
            sl = slice(256 * c, 256 * (c + 1))
            agg_ref[s, :, sl] = jnp.dot(
                Xp[:, sl], b_ref[s],
                preferred_element_type=jnp.float32).astype(jnp.bfloat16)
    w = w_ref[...]
    bias = bias_ref[...]
    mask = mask_ref[...]
    for n in range(NB):
        a = jnp.concatenate(
            [agg_ref[s, C * n:C * (n + 1), :] for s in range(_S)], axis=0)
        y = jnp.dot(w, a, preferred_element_type=jnp.float32) + bias
        y_ref[n] = y
        ym = y * mask
        s1_ref[n] = jnp.sum(ym, axis=1, keepdims=True)
        s2_ref[n] = jnp.sum(ym * y, axis=1, keepdims=True)


def _bn_kernel(y_ref, s1_ref, s2_ref, g_ref, be_ref, o_ref):
    NB, O, V, T = o_ref.shape
    TV = T * V
    N = s1_ref.shape[0]
    cnt = float(N * TV)
    mu = jnp.sum(s1_ref[...], axis=0) / cnt
    ex2 = jnp.sum(s2_ref[...], axis=0) / cnt
    var = jnp.maximum(ex2 - mu * mu, 0.0)
    inv = jax.lax.rsqrt(var + 1e-5)
    gcol = jnp.transpose(g_ref[...])
    bcol = jnp.transpose(be_ref[...])
    sc = gcol * inv
    sh = bcol - mu * sc
    n_out_groups = -(-TV // _GL)
    for n in range(NB):
        z = jnp.maximum(y_ref[n] * sc + sh, 0.0)
        parts = []
        for g in range(n_out_groups):
            w = min(_GL, TV - _GL * g)
            part = z[:, 128 * g:128 * g + w]
            if w < _GL:
                part = jnp.pad(part, ((0, 0), (0, _GL - w)))
            parts.append(part)
        zt = jnp.concatenate(parts, axis=1)[:, :TV]
        o_ref[n] = jnp.transpose(zt.reshape(O, T, V), (0, 2, 1))


def kernel(x, A_eff, w_conv, b_conv, gamma, beta):
    N, C, V, T = x.shape
    S = _S
    O = w_conv.shape[0]
    G = -(-T // _GF)
    if G % 2:
        G += 1
    L = G * 128
    TV = T * V

    A3 = A_eff.reshape(S, V, V)
    AT = jnp.swapaxes(A3, 1, 2)
    B = jnp.einsum('ab,suv->saubv', jnp.eye(2 * _GF, dtype=A_eff.dtype), AT)
    B = B.reshape(S, 2 * _GL, 2 * _GL)
    B = jnp.pad(B.reshape(S, 2, _GL, 2 * _GL),
                ((0, 0), (0, 0), (0, 128 - _GL), (0, 0)))
    B = B.reshape(S, 256, 2, _GL)
    B = jnp.pad(B, ((0, 0), (0, 0), (0, 0), (0, 128 - _GL)))
    B = B.reshape(S, 256, 256).astype(jnp.bfloat16)

    Wm = w_conv.astype(jnp.bfloat16)
    b2 = b_conv.reshape(O, 1).astype(jnp.float32)
    mask = _pack_lanes_const(jnp.ones((1, TV), jnp.float32))
    mask = jnp.pad(mask, ((0, 0), (0, L - mask.shape[-1])))

    y_pre, s1, s2 = pl.pallas_call(
        _msg_kernel,
        out_shape=(jax.ShapeDtypeStruct((N, O, L), jnp.float32),
                   jax.ShapeDtypeStruct((N, O, 1), jnp.float32),
                   jax.ShapeDtypeStruct((N, O, 1), jnp.float32)),
        grid=(N // _NB,),
        in_specs=[pl.BlockSpec((_NB, C, V, T), lambda i: (i, 0, 0, 0)),
                  pl.BlockSpec((S, 256, 256), lambda i: (0, 0, 0)),
                  pl.BlockSpec((O, S * C), lambda i: (0, 0)),
                  pl.BlockSpec((O, 1), lambda i: (0, 0)),
                  pl.BlockSpec((1, L), lambda i: (0, 0))],
        out_specs=(pl.BlockSpec((_NB, O, L), lambda i: (i, 0, 0)),
                   pl.BlockSpec((_NB, O, 1), lambda i: (i, 0, 0)),
                   pl.BlockSpec((_NB, O, 1), lambda i: (i, 0, 0))),
        scratch_shapes=[pltpu.VMEM((_NB * C, L), jnp.bfloat16),
                        pltpu.VMEM((S, _NB * C, L), jnp.bfloat16)],
        compiler_params=pltpu.CompilerParams(
            dimension_semantics=("parallel",),
            vmem_limit_bytes=64 * 1024 * 1024),
    )(x, B, Wm, b2, mask)

    out = pl.pallas_call(
        _bn_kernel,
        out_shape=jax.ShapeDtypeStruct((N, O, V, T), jnp.float32),
        grid=(N // _NB,),
        in_specs=[pl.BlockSpec((_NB, O, L), lambda i: (i, 0, 0)),
                  pl.BlockSpec((N, O, 1), lambda i: (0, 0, 0)),
                  pl.BlockSpec((N, O, 1), lambda i: (0, 0, 0)),
                  pl.BlockSpec((1, O), lambda i: (0, 0)),
                  pl.BlockSpec((1, O), lambda i: (0, 0))],
        out_specs=pl.BlockSpec((_NB, O, V, T), lambda i: (i, 0, 0, 0)),
        compiler_params=pltpu.CompilerParams(
            dimension_semantics=("parallel",)),
    )(y_pre, s1, s2, gamma.reshape(1, O), beta.reshape(1, O))

    return jnp.transpose(out, (0, 1, 3, 2))


# speedup vs baseline: 1.1875x; 1.1875x over previous
import jax
import jax.numpy as jnp
from jax.experimental import pallas as pl
from jax.experimental.pallas import tpu as pltpu

_S = 3
_V = 25
_GF = 5
_GL = _GF * _V
_NB = 4


def _pack_lanes_const(a):
    _, TV = a.shape
    G = -(-TV // _GL)
    a = jnp.pad(a, ((0, 0), (0, G * _GL - TV)))
    a = a.reshape(1, G, _GL)
    a = jnp.pad(a, ((0, 0), (0, 0), (0, 128 - _GL)))
    return a.reshape(1, G * 128)


def _msg_kernel(xt_ref, b_ref, w_ref, bias_ref, mask_ref,
                y_ref, s1_ref, s2_ref, xp_ref, agg_ref):
    NB, C, TV = xt_ref.shape
    L = xp_ref.shape[-1]
    X = xt_ref[...].reshape(NB * C, TV)
    ngo = -(-TV // _GL)
    for g in range(ngo):
        w = min(_GL, TV - _GL * g)
        xp_ref[:, 128 * g:128 * g + w] = \
            X[:, _GL * g:_GL * g + w].astype(jnp.bfloat16)
        xp_ref[:, 128 * g + w:128 * (g + 1)] = \
            jnp.zeros((NB * C, 128 - w), jnp.bfloat16)
    for g in range(ngo, L // 128):
        xp_ref[:, 128 * g:128 * (g + 1)] = jnp.zeros((NB * C, 128),
                                                     jnp.bfloat16)
    Xp = xp_ref[...]
    for s in range(_S):
        for c in range(L // 256):
            sl = slice(256 * c, 256 * (c + 1))
            agg_ref[s, :, sl] = jnp.dot(
                Xp[:, sl], b_ref[s],
                preferred_element_type=jnp.float32).astype(jnp.bfloat16)
    w = w_ref[...]
    bias = bias_ref[...]
    mask = mask_ref[...]
    for n in range(NB):
        a = jnp.concatenate(
            [agg_ref[s, C * n:C * (n + 1), :] for s in range(_S)], axis=0)
        y = jnp.dot(w, a, preferred_element_type=jnp.float32) + bias
        y_ref[n] = y.astype(jnp.bfloat16)
        ym = y * mask
        s1_ref[n] = jnp.sum(ym, axis=1, keepdims=True)
        s2_ref[n] = jnp.sum(ym * y, axis=1, keepdims=True)


def _bn_kernel(y_ref, s1_ref, s2_ref, g_ref, be_ref, o_ref):
    NB, O, TV = o_ref.shape
    N = s1_ref.shape[0]
    cnt = float(N * TV)
    mu = jnp.sum(s1_ref[...], axis=0) / cnt
    ex2 = jnp.sum(s2_ref[...], axis=0) / cnt
    var = jnp.maximum(ex2 - mu * mu, 0.0)
    inv = jax.lax.rsqrt(var + 1e-5)
    gcol = jnp.transpose(g_ref[...])
    bcol = jnp.transpose(be_ref[...])
    sc = gcol * inv
    sh = bcol - mu * sc
    n_out_groups = -(-TV // _GL)
    for n in range(NB):
        z = jnp.maximum(y_ref[n].astype(jnp.float32) * sc + sh, 0.0)
        for g in range(n_out_groups):
            w = min(_GL, TV - _GL * g)
            o_ref[n, :, _GL * g:_GL * g + w] = z[:, 128 * g:128 * g + w]


def kernel(x, A_eff, w_conv, b_conv, gamma, beta):
    N, C, V, T = x.shape
    S = _S
    O = w_conv.shape[0]
    G = -(-T // _GF)
    if G % 2:
        G += 1
    L = G * 128
    TV = T * V

    xt = jnp.transpose(x, (0, 1, 3, 2)).reshape(N, C, TV)

    A3 = A_eff.reshape(S, V, V)
    AT = jnp.swapaxes(A3, 1, 2)
    B = jnp.einsum('ab,suv->saubv', jnp.eye(2 * _GF, dtype=A_eff.dtype), AT)
    B = B.reshape(S, 2 * _GL, 2 * _GL)
    B = jnp.pad(B.reshape(S, 2, _GL, 2 * _GL),
                ((0, 0), (0, 0), (0, 128 - _GL), (0, 0)))
    B = B.reshape(S, 256, 2, _GL)
    B = jnp.pad(B, ((0, 0), (0, 0), (0, 0), (0, 128 - _GL)))
    B = B.reshape(S, 256, 256).astype(jnp.bfloat16)

    Wm = w_conv.astype(jnp.bfloat16)
    b2 = b_conv.reshape(O, 1).astype(jnp.float32)
    mask = _pack_lanes_const(jnp.ones((1, TV), jnp.float32))
    mask = jnp.pad(mask, ((0, 0), (0, L - mask.shape[-1])))

    y_pre, s1, s2 = pl.pallas_call(
        _msg_kernel,
        out_shape=(jax.ShapeDtypeStruct((N, O, L), jnp.bfloat16),
                   jax.ShapeDtypeStruct((N, O, 1), jnp.float32),
                   jax.ShapeDtypeStruct((N, O, 1), jnp.float32)),
        grid=(N // _NB,),
        in_specs=[pl.BlockSpec((_NB, C, TV), lambda i: (i, 0, 0)),
                  pl.BlockSpec((S, 256, 256), lambda i: (0, 0, 0)),
                  pl.BlockSpec((O, S * C), lambda i: (0, 0)),
                  pl.BlockSpec((O, 1), lambda i: (0, 0)),
                  pl.BlockSpec((1, L), lambda i: (0, 0))],
        out_specs=(pl.BlockSpec((_NB, O, L), lambda i: (i, 0, 0)),
                   pl.BlockSpec((_NB, O, 1), lambda i: (i, 0, 0)),
                   pl.BlockSpec((_NB, O, 1), lambda i: (i, 0, 0))),
        scratch_shapes=[pltpu.VMEM((_NB * C, L), jnp.bfloat16),
                        pltpu.VMEM((S, _NB * C, L), jnp.bfloat16)],
        compiler_params=pltpu.CompilerParams(
            dimension_semantics=("parallel",),
            vmem_limit_bytes=64 * 1024 * 1024),
    )(xt, B, Wm, b2, mask)

    out = pl.pallas_call(
        _bn_kernel,
        out_shape=jax.ShapeDtypeStruct((N, O, TV), jnp.float32),
        grid=(N // _NB,),
        in_specs=[pl.BlockSpec((_NB, O, L), lambda i: (i, 0, 0)),
                  pl.BlockSpec((N, O, 1), lambda i: (0, 0, 0)),
                  pl.BlockSpec((N, O, 1), lambda i: (0, 0, 0)),
                  pl.BlockSpec((1, O), lambda i: (0, 0)),
                  pl.BlockSpec((1, O), lambda i: (0, 0))],
        out_specs=pl.BlockSpec((_NB, O, TV), lambda i: (i, 0, 0)),
        compiler_params=pltpu.CompilerParams(
            dimension_semantics=("parallel",)),
    )(y_pre, s1, s2, gamma.reshape(1, O), beta.reshape(1, O))

    return out.reshape(N, O, T, V)


# speedup vs baseline: 1.1932x; 1.0048x over previous
import jax
import jax.numpy as jnp
from jax.experimental import pallas as pl
from jax.experimental.pallas import tpu as pltpu

_S = 3
_V = 25
_GF = 5
_GL = _GF * _V
_NB = 4


def _pack_lanes_const(a):
    _, TV = a.shape
    G = -(-TV // _GL)
    a = jnp.pad(a, ((0, 0), (0, G * _GL - TV)))
    a = a.reshape(1, G, _GL)
    a = jnp.pad(a, ((0, 0), (0, 0), (0, 128 - _GL)))
    return a.reshape(1, G * 128)


def _msg_kernel(xt_ref, b_ref, w_ref, bias_ref, mask_ref,
                y_ref, s1_ref, s2_ref, xp_ref, agg_ref):
    NB, C, TV = xt_ref.shape
    L = xp_ref.shape[-1]
    X = xt_ref[...].reshape(NB * C, TV)
    ngo = -(-TV // _GL)
    for g in range(ngo):
        w = min(_GL, TV - _GL * g)
        xp_ref[:, 128 * g:128 * g + w] = X[:, _GL * g:_GL * g + w]
        xp_ref[:, 128 * g + w:128 * (g + 1)] = \
            jnp.zeros((NB * C, 128 - w), jnp.bfloat16)
    for g in range(ngo, L // 128):
        xp_ref[:, 128 * g:128 * (g + 1)] = jnp.zeros((NB * C, 128),
                                                     jnp.bfloat16)
    Xp = xp_ref[...]
    for s in range(_S):
        for c in range(L // 256):
            sl = slice(256 * c, 256 * (c + 1))
            agg_ref[s, :, sl] = jnp.dot(
                Xp[:, sl], b_ref[s],
                preferred_element_type=jnp.float32).astype(jnp.bfloat16)
    w = w_ref[...]
    bias = bias_ref[...]
    mask = mask_ref[...]
    for n in range(NB):
        a = jnp.concatenate(
            [agg_ref[s, C * n:C * (n + 1), :] for s in range(_S)], axis=0)
        y = jnp.dot(w, a, preferred_element_type=jnp.float32) + bias
        y_ref[n] = y.astype(jnp.bfloat16)
        ym = y * mask
        s1_ref[n] = jnp.sum(ym, axis=1, keepdims=True)
        s2_ref[n] = jnp.sum(ym * y, axis=1, keepdims=True)


def _bn_kernel(y_ref, s1_ref, s2_ref, g_ref, be_ref, o_ref):
    NB, O, TV = o_ref.shape
    N = s1_ref.shape[0]
    cnt = float(N * TV)
    mu = jnp.sum(s1_ref[...], axis=0) / cnt
    ex2 = jnp.sum(s2_ref[...], axis=0) / cnt
    var = jnp.maximum(ex2 - mu * mu, 0.0)
    inv = jax.lax.rsqrt(var + 1e-5)
    gcol = jnp.transpose(g_ref[...])
    bcol = jnp.transpose(be_ref[...])
    sc = gcol * inv
    sh = bcol - mu * sc
    n_out_groups = -(-TV // _GL)
    for n in range(NB):
        z = jnp.maximum(y_ref[n].astype(jnp.float32) * sc + sh, 0.0)
        for g in range(n_out_groups):
            w = min(_GL, TV - _GL * g)
            o_ref[n, :, _GL * g:_GL * g + w] = z[:, 128 * g:128 * g + w]


def kernel(x, A_eff, w_conv, b_conv, gamma, beta):
    N, C, V, T = x.shape
    S = _S
    O = w_conv.shape[0]
    G = -(-T // _GF)
    if G % 2:
        G += 1
    L = G * 128
    TV = T * V

    xt = jnp.transpose(x.astype(jnp.bfloat16),
                       (0, 1, 3, 2)).reshape(N, C, TV)

    A3 = A_eff.reshape(S, V, V)
    AT = jnp.swapaxes(A3, 1, 2)
    B = jnp.einsum('ab,suv->saubv', jnp.eye(2 * _GF, dtype=A_eff.dtype), AT)
    B = B.reshape(S, 2 * _GL, 2 * _GL)
    B = jnp.pad(B.reshape(S, 2, _GL, 2 * _GL),
                ((0, 0), (0, 0), (0, 128 - _GL), (0, 0)))
    B = B.reshape(S, 256, 2, _GL)
    B = jnp.pad(B, ((0, 0), (0, 0), (0, 0), (0, 128 - _GL)))
    B = B.reshape(S, 256, 256).astype(jnp.bfloat16)

    Wm = w_conv.astype(jnp.bfloat16)
    b2 = b_conv.reshape(O, 1).astype(jnp.float32)
    mask = _pack_lanes_const(jnp.ones((1, TV), jnp.float32))
    mask = jnp.pad(mask, ((0, 0), (0, L - mask.shape[-1])))

    y_pre, s1, s2 = pl.pallas_call(
        _msg_kernel,
        out_shape=(jax.ShapeDtypeStruct((N, O, L), jnp.bfloat16),
                   jax.ShapeDtypeStruct((N, O, 1), jnp.float32),
                   jax.ShapeDtypeStruct((N, O, 1), jnp.float32)),
        grid=(N // _NB,),
        in_specs=[pl.BlockSpec((_NB, C, TV), lambda i: (i, 0, 0)),
                  pl.BlockSpec((S, 256, 256), lambda i: (0, 0, 0)),
                  pl.BlockSpec((O, S * C), lambda i: (0, 0)),
                  pl.BlockSpec((O, 1), lambda i: (0, 0)),
                  pl.BlockSpec((1, L), lambda i: (0, 0))],
        out_specs=(pl.BlockSpec((_NB, O, L), lambda i: (i, 0, 0)),
                   pl.BlockSpec((_NB, O, 1), lambda i: (i, 0, 0)),
                   pl.BlockSpec((_NB, O, 1), lambda i: (i, 0, 0))),
        scratch_shapes=[pltpu.VMEM((_NB * C, L), jnp.bfloat16),
                        pltpu.VMEM((S, _NB * C, L), jnp.bfloat16)],
        compiler_params=pltpu.CompilerParams(
            dimension_semantics=("parallel",),
            vmem_limit_bytes=64 * 1024 * 1024),
    )(xt, B, Wm, b2, mask)

    out = pl.pallas_call(
        _bn_kernel,
        out_shape=jax.ShapeDtypeStruct((N, O, TV), jnp.float32),
        grid=(N // _NB,),
        in_specs=[pl.BlockSpec((_NB, O, L), lambda i: (i, 0, 0)),
                  pl.BlockSpec((N, O, 1), lambda i: (0, 0, 0)),
                  pl.BlockSpec((N, O, 1), lambda i: (0, 0, 0)),
                  pl.BlockSpec((1, O), lambda i: (0, 0)),
                  pl.BlockSpec((1, O), lambda i: (0, 0))],
        out_specs=pl.BlockSpec((_NB, O, TV), lambda i: (i, 0, 0)),
        compiler_params=pltpu.CompilerParams(
            dimension_semantics=("parallel",)),
    )(y_pre, s1, s2, gamma.reshape(1, O), beta.reshape(1, O))

    return out.reshape(N, O, T, V)


# speedup vs baseline: 1.2347x; 1.0348x over previous
import jax
import jax.numpy as jnp
from jax.experimental import pallas as pl
from jax.experimental.pallas import tpu as pltpu

_S = 3
_V = 25
_GF = 5
_GL = _GF * _V
_NB = 8


def _pack_lanes_const(a):
    _, TV = a.shape
    G = -(-TV // _GL)
    a = jnp.pad(a, ((0, 0), (0, G * _GL - TV)))
    a = a.reshape(1, G, _GL)
    a = jnp.pad(a, ((0, 0), (0, 0), (0, 128 - _GL)))
    return a.reshape(1, G * 128)


def _msg_kernel(xt_ref, b_ref, w_ref, bias_ref, mask_ref,
                y_ref, s1_ref, s2_ref, xp_ref, agg_ref):
    NB, C, TV = xt_ref.shape
    L = xp_ref.shape[-1]
    X = xt_ref[...].reshape(NB * C, TV)
    ngo = -(-TV // _GL)
    for g in range(ngo):
        w = min(_GL, TV - _GL * g)
        xp_ref[:, 128 * g:128 * g + w] = X[:, _GL * g:_GL * g + w]
        xp_ref[:, 128 * g + w:128 * (g + 1)] = \
            jnp.zeros((NB * C, 128 - w), jnp.bfloat16)
    for g in range(ngo, L // 128):
        xp_ref[:, 128 * g:128 * (g + 1)] = jnp.zeros((NB * C, 128),
                                                     jnp.bfloat16)
    Xp = xp_ref[...]
    for s in range(_S):
        for c in range(L // 256):
            sl = slice(256 * c, 256 * (c + 1))
            agg_ref[s, :, sl] = jnp.dot(
                Xp[:, sl], b_ref[s],
                preferred_element_type=jnp.float32).astype(jnp.bfloat16)
    w = w_ref[...]
    bias = bias_ref[...]
    mask = mask_ref[...]
    for n in range(NB):
        a = jnp.concatenate(
            [agg_ref[s, C * n:C * (n + 1), :] for s in range(_S)], axis=0)
        y = jnp.dot(w, a, preferred_element_type=jnp.float32) + bias
        y_ref[n] = y.astype(jnp.bfloat16)
        ym = y * mask
        s1_ref[n] = jnp.sum(ym, axis=1, keepdims=True)
        s2_ref[n] = jnp.sum(ym * y, axis=1, keepdims=True)


def _bn_kernel(y_ref, s1_ref, s2_ref, g_ref, be_ref, o_ref):
    NB, O, TV = o_ref.shape
    N = s1_ref.shape[0]
    cnt = float(N * TV)
    mu = jnp.sum(s1_ref[...], axis=0) / cnt
    ex2 = jnp.sum(s2_ref[...], axis=0) / cnt
    var = jnp.maximum(ex2 - mu * mu, 0.0)
    inv = jax.lax.rsqrt(var + 1e-5)
    gcol = jnp.transpose(g_ref[...])
    bcol = jnp.transpose(be_ref[...])
    sc = gcol * inv
    sh = bcol - mu * sc
    n_out_groups = -(-TV // _GL)
    for n in range(NB):
        z = jnp.maximum(y_ref[n].astype(jnp.float32) * sc + sh, 0.0)
        for g in range(n_out_groups):
            w = min(_GL, TV - _GL * g)
            o_ref[n, :, _GL * g:_GL * g + w] = z[:, 128 * g:128 * g + w]


def kernel(x, A_eff, w_conv, b_conv, gamma, beta):
    N, C, V, T = x.shape
    S = _S
    O = w_conv.shape[0]
    G = -(-T // _GF)
    if G % 2:
        G += 1
    L = G * 128
    TV = T * V

    xt = jnp.transpose(x.astype(jnp.bfloat16),
                       (0, 1, 3, 2)).reshape(N, C, TV)

    A3 = A_eff.reshape(S, V, V)
    AT = jnp.swapaxes(A3, 1, 2)
    B = jnp.einsum('ab,suv->saubv', jnp.eye(2 * _GF, dtype=A_eff.dtype), AT)
    B = B.reshape(S, 2 * _GL, 2 * _GL)
    B = jnp.pad(B.reshape(S, 2, _GL, 2 * _GL),
                ((0, 0), (0, 0), (0, 128 - _GL), (0, 0)))
    B = B.reshape(S, 256, 2, _GL)
    B = jnp.pad(B, ((0, 0), (0, 0), (0, 0), (0, 128 - _GL)))
    B = B.reshape(S, 256, 256).astype(jnp.bfloat16)

    Wm = w_conv.astype(jnp.bfloat16)
    b2 = b_conv.reshape(O, 1).astype(jnp.float32)
    mask = _pack_lanes_const(jnp.ones((1, TV), jnp.float32))
    mask = jnp.pad(mask, ((0, 0), (0, L - mask.shape[-1])))

    y_pre, s1, s2 = pl.pallas_call(
        _msg_kernel,
        out_shape=(jax.ShapeDtypeStruct((N, O, L), jnp.bfloat16),
                   jax.ShapeDtypeStruct((N, O, 1), jnp.float32),
                   jax.ShapeDtypeStruct((N, O, 1), jnp.float32)),
        grid=(N // _NB,),
        in_specs=[pl.BlockSpec((_NB, C, TV), lambda i: (i, 0, 0)),
                  pl.BlockSpec((S, 256, 256), lambda i: (0, 0, 0)),
                  pl.BlockSpec((O, S * C), lambda i: (0, 0)),
                  pl.BlockSpec((O, 1), lambda i: (0, 0)),
                  pl.BlockSpec((1, L), lambda i: (0, 0))],
        out_specs=(pl.BlockSpec((_NB, O, L), lambda i: (i, 0, 0)),
                   pl.BlockSpec((_NB, O, 1), lambda i: (i, 0, 0)),
                   pl.BlockSpec((_NB, O, 1), lambda i: (i, 0, 0))),
        scratch_shapes=[pltpu.VMEM((_NB * C, L), jnp.bfloat16),
                        pltpu.VMEM((S, _NB * C, L), jnp.bfloat16)],
        compiler_params=pltpu.CompilerParams(
            dimension_semantics=("parallel",),
            vmem_limit_bytes=64 * 1024 * 1024),
    )(xt, B, Wm, b2, mask)

    out = pl.pallas_call(
        _bn_kernel,
        out_shape=jax.ShapeDtypeStruct((N, O, TV), jnp.float32),
        grid=(N // _NB,),
        in_specs=[pl.BlockSpec((_NB, O, L), lambda i: (i, 0, 0)),
                  pl.BlockSpec((N, O, 1), lambda i: (0, 0, 0)),
                  pl.BlockSpec((N, O, 1), lambda i: (0, 0, 0)),
                  pl.BlockSpec((1, O), lambda i: (0, 0)),
                  pl.BlockSpec((1, O), lambda i: (0, 0))],
        out_specs=pl.BlockSpec((_NB, O, TV), lambda i: (i, 0, 0)),
        compiler_params=pltpu.CompilerParams(
            dimension_semantics=("parallel",)),
    )(y_pre, s1, s2, gamma.reshape(1, O), beta.reshape(1, O))

    return out.reshape(N, O, T, V)


# speedup vs baseline: 1.2487x; 1.0114x over previous
import jax
import jax.numpy as jnp
from jax.experimental import pallas as pl
from jax.experimental.pallas import tpu as pltpu

_S = 3
_V = 25
_GF = 5
_GL = _GF * _V
_NB = 16


def _pack_lanes_const(a):
    _, TV = a.shape
    G = -(-TV // _GL)
    a = jnp.pad(a, ((0, 0), (0, G * _GL - TV)))
    a = a.reshape(1, G, _GL)
    a = jnp.pad(a, ((0, 0), (0, 0), (0, 128 - _GL)))
    return a.reshape(1, G * 128)


def _msg_kernel(xt_ref, b_ref, w_ref, bias_ref, mask_ref,
                y_ref, s1_ref, s2_ref, xp_ref, agg_ref):
    NB, C, TV = xt_ref.shape
    L = xp_ref.shape[-1]
    X = xt_ref[...].reshape(NB * C, TV)
    ngo = -(-TV // _GL)
    for g in range(ngo):
        w = min(_GL, TV - _GL * g)
        xp_ref[:, 128 * g:128 * g + w] = X[:, _GL * g:_GL * g + w]
        xp_ref[:, 128 * g + w:128 * (g + 1)] = \
            jnp.zeros((NB * C, 128 - w), jnp.bfloat16)
    for g in range(ngo, L // 128):
        xp_ref[:, 128 * g:128 * (g + 1)] = jnp.zeros((NB * C, 128),
                                                     jnp.bfloat16)
    Xp = xp_ref[...]
    for s in range(_S):
        for c in range(L // 256):
            sl = slice(256 * c, 256 * (c + 1))
            agg_ref[s, :, sl] = jnp.dot(
                Xp[:, sl], b_ref[s],
                preferred_element_type=jnp.float32).astype(jnp.bfloat16)
    w = w_ref[...]
    bias = bias_ref[...]
    mask = mask_ref[...]
    for n in range(NB):
        a = jnp.concatenate(
            [agg_ref[s, C * n:C * (n + 1), :] for s in range(_S)], axis=0)
        y = jnp.dot(w, a, preferred_element_type=jnp.float32) + bias
        y_ref[n] = y.astype(jnp.bfloat16)
        ym = y * mask
        s1_ref[n] = jnp.sum(ym, axis=1, keepdims=True)
        s2_ref[n] = jnp.sum(ym * y, axis=1, keepdims=True)


def _bn_kernel(y_ref, s1_ref, s2_ref, g_ref, be_ref, o_ref):
    NB, O, TV = o_ref.shape
    N = s1_ref.shape[0]
    cnt = float(N * TV)
    mu = jnp.sum(s1_ref[...], axis=0) / cnt
    ex2 = jnp.sum(s2_ref[...], axis=0) / cnt
    var = jnp.maximum(ex2 - mu * mu, 0.0)
    inv = jax.lax.rsqrt(var + 1e-5)
    gcol = jnp.transpose(g_ref[...])
    bcol = jnp.transpose(be_ref[...])
    sc = gcol * inv
    sh = bcol - mu * sc
    n_out_groups = -(-TV // _GL)
    for n in range(NB):
        z = jnp.maximum(y_ref[n].astype(jnp.float32) * sc + sh, 0.0)
        for g in range(n_out_groups):
            w = min(_GL, TV - _GL * g)
            o_ref[n, :, _GL * g:_GL * g + w] = z[:, 128 * g:128 * g + w]


def kernel(x, A_eff, w_conv, b_conv, gamma, beta):
    N, C, V, T = x.shape
    S = _S
    O = w_conv.shape[0]
    G = -(-T // _GF)
    if G % 2:
        G += 1
    L = G * 128
    TV = T * V

    xt = jnp.transpose(x.astype(jnp.bfloat16),
                       (0, 1, 3, 2)).reshape(N, C, TV)

    A3 = A_eff.reshape(S, V, V)
    AT = jnp.swapaxes(A3, 1, 2)
    B = jnp.einsum('ab,suv->saubv', jnp.eye(2 * _GF, dtype=A_eff.dtype), AT)
    B = B.reshape(S, 2 * _GL, 2 * _GL)
    B = jnp.pad(B.reshape(S, 2, _GL, 2 * _GL),
                ((0, 0), (0, 0), (0, 128 - _GL), (0, 0)))
    B = B.reshape(S, 256, 2, _GL)
    B = jnp.pad(B, ((0, 0), (0, 0), (0, 0), (0, 128 - _GL)))
    B = B.reshape(S, 256, 256).astype(jnp.bfloat16)

    Wm = w_conv.astype(jnp.bfloat16)
    b2 = b_conv.reshape(O, 1).astype(jnp.float32)
    mask = _pack_lanes_const(jnp.ones((1, TV), jnp.float32))
    mask = jnp.pad(mask, ((0, 0), (0, L - mask.shape[-1])))

    y_pre, s1, s2 = pl.pallas_call(
        _msg_kernel,
        out_shape=(jax.ShapeDtypeStruct((N, O, L), jnp.bfloat16),
                   jax.ShapeDtypeStruct((N, O, 1), jnp.float32),
                   jax.ShapeDtypeStruct((N, O, 1), jnp.float32)),
        grid=(N // _NB,),
        in_specs=[pl.BlockSpec((_NB, C, TV), lambda i: (i, 0, 0)),
                  pl.BlockSpec((S, 256, 256), lambda i: (0, 0, 0)),
                  pl.BlockSpec((O, S * C), lambda i: (0, 0)),
                  pl.BlockSpec((O, 1), lambda i: (0, 0)),
                  pl.BlockSpec((1, L), lambda i: (0, 0))],
        out_specs=(pl.BlockSpec((_NB, O, L), lambda i: (i, 0, 0)),
                   pl.BlockSpec((_NB, O, 1), lambda i: (i, 0, 0)),
                   pl.BlockSpec((_NB, O, 1), lambda i: (i, 0, 0))),
        scratch_shapes=[pltpu.VMEM((_NB * C, L), jnp.bfloat16),
                        pltpu.VMEM((S, _NB * C, L), jnp.bfloat16)],
        compiler_params=pltpu.CompilerParams(
            dimension_semantics=("parallel",),
            vmem_limit_bytes=64 * 1024 * 1024),
    )(xt, B, Wm, b2, mask)

    out = pl.pallas_call(
        _bn_kernel,
        out_shape=jax.ShapeDtypeStruct((N, O, TV), jnp.float32),
        grid=(N // _NB,),
        in_specs=[pl.BlockSpec((_NB, O, L), lambda i: (i, 0, 0)),
                  pl.BlockSpec((N, O, 1), lambda i: (0, 0, 0)),
                  pl.BlockSpec((N, O, 1), lambda i: (0, 0, 0)),
                  pl.BlockSpec((1, O), lambda i: (0, 0)),
                  pl.BlockSpec((1, O), lambda i: (0, 0))],
        out_specs=pl.BlockSpec((_NB, O, TV), lambda i: (i, 0, 0)),
        compiler_params=pltpu.CompilerParams(
            dimension_semantics=("parallel",)),
    )(y_pre, s1, s2, gamma.reshape(1, O), beta.reshape(1, O))

    return out.reshape(N, O, T, V)


# speedup vs baseline: 1.2545x; 1.0047x over previous
import jax
import jax.numpy as jnp
from jax.experimental import pallas as pl
from jax.experimental.pallas import tpu as pltpu

_S = 3
_V = 25
_GF = 5
_GL = _GF * _V
_NB = 16


def _pack_lanes_const(a):
    _, TV = a.shape
    G = -(-TV // _GL)
    a = jnp.pad(a, ((0, 0), (0, G * _GL - TV)))
    a = a.reshape(1, G, _GL)
    a = jnp.pad(a, ((0, 0), (0, 0), (0, 128 - _GL)))
    return a.reshape(1, G * 128)


def _msg_kernel(xt_ref, b_ref, w_ref, bias_ref, mask_ref,
                y_ref, s1_ref, s2_ref, xp_ref, agg_ref):
    NB, C, TV = xt_ref.shape
    L = xp_ref.shape[-1]
    X = xt_ref[...].reshape(NB * C, TV)
    ngo = -(-TV // _GL)
    for g in range(ngo):
        w = min(_GL, TV - _GL * g)
        xp_ref[:, 128 * g:128 * g + w] = \
            X[:, _GL * g:_GL * g + w].astype(jnp.bfloat16)
        xp_ref[:, 128 * g + w:128 * (g + 1)] = \
            jnp.zeros((NB * C, 128 - w), jnp.bfloat16)
    for g in range(ngo, L // 128):
        xp_ref[:, 128 * g:128 * (g + 1)] = jnp.zeros((NB * C, 128),
                                                     jnp.bfloat16)
    Xp = xp_ref[...]
    for s in range(_S):
        for c in range(L // 256):
            sl = slice(256 * c, 256 * (c + 1))
            agg_ref[s, :, sl] = jnp.dot(
                Xp[:, sl], b_ref[s],
                preferred_element_type=jnp.float32).astype(jnp.bfloat16)
    w = w_ref[...]
    bias = bias_ref[...]
    mask = mask_ref[...]
    for n in range(NB):
        a = jnp.concatenate(
            [agg_ref[s, C * n:C * (n + 1), :] for s in range(_S)], axis=0)
        y = jnp.dot(w, a, preferred_element_type=jnp.float32) + bias
        y_ref[n] = y.astype(jnp.bfloat16)
        ym = y * mask
        s1_ref[n] = jnp.sum(ym, axis=1, keepdims=True)
        s2_ref[n] = jnp.sum(ym * y, axis=1, keepdims=True)


def _bn_kernel(y_ref, s1_ref, s2_ref, g_ref, be_ref, o_ref):
    NB, O, TV = o_ref.shape
    N = s1_ref.shape[0]
    cnt = float(N * TV)
    mu = jnp.sum(s1_ref[...], axis=0) / cnt
    ex2 = jnp.sum(s2_ref[...], axis=0) / cnt
    var = jnp.maximum(ex2 - mu * mu, 0.0)
    inv = jax.lax.rsqrt(var + 1e-5)
    gcol = jnp.transpose(g_ref[...])
    bcol = jnp.transpose(be_ref[...])
    sc = gcol * inv
    sh = bcol - mu * sc
    n_out_groups = -(-TV // _GL)
    for n in range(NB):
        z = jnp.maximum(y_ref[n].astype(jnp.float32) * sc + sh, 0.0)
        for g in range(n_out_groups):
            w = min(_GL, TV - _GL * g)
            o_ref[n, :, _GL * g:_GL * g + w] = z[:, 128 * g:128 * g + w]


def kernel(x, A_eff, w_conv, b_conv, gamma, beta):
    N, C, V, T = x.shape
    S = _S
    O = w_conv.shape[0]
    G = -(-T // _GF)
    if G % 2:
        G += 1
    L = G * 128
    TV = T * V

    xt = jnp.transpose(x, (0, 1, 3, 2)).reshape(N, C, TV)

    A3 = A_eff.reshape(S, V, V)
    AT = jnp.swapaxes(A3, 1, 2)
    B = jnp.einsum('ab,suv->saubv', jnp.eye(2 * _GF, dtype=A_eff.dtype), AT)
    B = B.reshape(S, 2 * _GL, 2 * _GL)
    B = jnp.pad(B.reshape(S, 2, _GL, 2 * _GL),
                ((0, 0), (0, 0), (0, 128 - _GL), (0, 0)))
    B = B.reshape(S, 256, 2, _GL)
    B = jnp.pad(B, ((0, 0), (0, 0), (0, 0), (0, 128 - _GL)))
    B = B.reshape(S, 256, 256).astype(jnp.bfloat16)

    Wm = w_conv.astype(jnp.bfloat16)
    b2 = b_conv.reshape(O, 1).astype(jnp.float32)
    mask = _pack_lanes_const(jnp.ones((1, TV), jnp.float32))
    mask = jnp.pad(mask, ((0, 0), (0, L - mask.shape[-1])))

    y_pre, s1, s2 = pl.pallas_call(
        _msg_kernel,
        out_shape=(jax.ShapeDtypeStruct((N, O, L), jnp.bfloat16),
                   jax.ShapeDtypeStruct((N, O, 1), jnp.float32),
                   jax.ShapeDtypeStruct((N, O, 1), jnp.float32)),
        grid=(N // _NB,),
        in_specs=[pl.BlockSpec((_NB, C, TV), lambda i: (i, 0, 0)),
                  pl.BlockSpec((S, 256, 256), lambda i: (0, 0, 0)),
                  pl.BlockSpec((O, S * C), lambda i: (0, 0)),
                  pl.BlockSpec((O, 1), lambda i: (0, 0)),
                  pl.BlockSpec((1, L), lambda i: (0, 0))],
        out_specs=(pl.BlockSpec((_NB, O, L), lambda i: (i, 0, 0)),
                   pl.BlockSpec((_NB, O, 1), lambda i: (i, 0, 0)),
                   pl.BlockSpec((_NB, O, 1), lambda i: (i, 0, 0))),
        scratch_shapes=[pltpu.VMEM((_NB * C, L), jnp.bfloat16),
                        pltpu.VMEM((S, _NB * C, L), jnp.bfloat16)],
        compiler_params=pltpu.CompilerParams(
            dimension_semantics=("parallel",),
            vmem_limit_bytes=64 * 1024 * 1024),
    )(xt, B, Wm, b2, mask)

    out = pl.pallas_call(
        _bn_kernel,
        out_shape=jax.ShapeDtypeStruct((N, O, TV), jnp.float32),
        grid=(N // _NB,),
        in_specs=[pl.BlockSpec((_NB, O, L), lambda i: (i, 0, 0)),
                  pl.BlockSpec((N, O, 1), lambda i: (0, 0, 0)),
                  pl.BlockSpec((N, O, 1), lambda i: (0, 0, 0)),
                  pl.BlockSpec((1, O), lambda i: (0, 0)),
                  pl.BlockSpec((1, O), lambda i: (0, 0))],
        out_specs=pl.BlockSpec((_NB, O, TV), lambda i: (i, 0, 0)),
        compiler_params=pltpu.CompilerParams(
            dimension_semantics=("parallel",)),
    )(y_pre, s1, s2, gamma.reshape(1, O), beta.reshape(1, O))

    return out.reshape(N, O, T, V)


# speedup vs baseline: 1.2644x; 1.0079x over previous
import jax
import jax.numpy as jnp
from jax.experimental import pallas as pl
from jax.experimental.pallas import tpu as pltpu

_S = 3
_V = 25
_GF = 5
_GL = _GF * _V
_NB = 16


def _msg_kernel(xt_ref, b_ref, w_ref,
                y_ref, s1_ref, s2_ref, xp_ref, agg_ref):
    NB, C, TV = xt_ref.shape
    L = xp_ref.shape[-1]
    X = xt_ref[...].reshape(NB * C, TV)
    ngo = -(-TV // _GL)
    for g in range(ngo):
        w = min(_GL, TV - _GL * g)
        xp_ref[:, 128 * g:128 * g + w] = \
            X[:, _GL * g:_GL * g + w].astype(jnp.bfloat16)
        xp_ref[:, 128 * g + w:128 * (g + 1)] = \
            jnp.zeros((NB * C, 128 - w), jnp.bfloat16)
    for g in range(ngo, L // 128):
        xp_ref[:, 128 * g:128 * (g + 1)] = jnp.zeros((NB * C, 128),
                                                     jnp.bfloat16)
    Xp = xp_ref[...]
    for s in range(_S):
        for c in range(L // 256):
            sl = slice(256 * c, 256 * (c + 1))
            agg_ref[s, :, sl] = jnp.dot(
                Xp[:, sl], b_ref[s],
                preferred_element_type=jnp.float32).astype(jnp.bfloat16)
    w = w_ref[...]
    for n in range(NB):
        a = jnp.concatenate(
            [agg_ref[s, C * n:C * (n + 1), :] for s in range(_S)], axis=0)
        y = jnp.dot(w, a, preferred_element_type=jnp.float32)
        y_ref[n] = y.astype(jnp.bfloat16)
        s1_ref[n] = jnp.sum(y, axis=1, keepdims=True)
        s2_ref[n] = jnp.sum(y * y, axis=1, keepdims=True)


def _bn_kernel(y_ref, s1_ref, s2_ref, g_ref, be_ref, o_ref):
    NB, O, TV = o_ref.shape
    N = s1_ref.shape[0]
    cnt = float(N * TV)
    md = jnp.sum(s1_ref[...], axis=0) / cnt
    ex2 = jnp.sum(s2_ref[...], axis=0) / cnt
    var = jnp.maximum(ex2 - md * md, 0.0)
    inv = jax.lax.rsqrt(var + 1e-5)
    gcol = jnp.transpose(g_ref[...])
    bcol = jnp.transpose(be_ref[...])
    sc = gcol * inv
    sh = bcol - md * sc
    n_out_groups = -(-TV // _GL)
    for n in range(NB):
        z = jnp.maximum(y_ref[n].astype(jnp.float32) * sc + sh, 0.0)
        for g in range(n_out_groups):
            w = min(_GL, TV - _GL * g)
            o_ref[n, :, _GL * g:_GL * g + w] = z[:, 128 * g:128 * g + w]


def kernel(x, A_eff, w_conv, b_conv, gamma, beta):
    N, C, V, T = x.shape
    S = _S
    O = w_conv.shape[0]
    G = -(-T // _GF)
    if G % 2:
        G += 1
    L = G * 128
    TV = T * V

    xt = jnp.transpose(x, (0, 1, 3, 2)).reshape(N, C, TV)

    A3 = A_eff.reshape(S, V, V)
    AT = jnp.swapaxes(A3, 1, 2)
    B = jnp.einsum('ab,suv->saubv', jnp.eye(2 * _GF, dtype=A_eff.dtype), AT)
    B = B.reshape(S, 2 * _GL, 2 * _GL)
    B = jnp.pad(B.reshape(S, 2, _GL, 2 * _GL),
                ((0, 0), (0, 0), (0, 128 - _GL), (0, 0)))
    B = B.reshape(S, 256, 2, _GL)
    B = jnp.pad(B, ((0, 0), (0, 0), (0, 0), (0, 128 - _GL)))
    B = B.reshape(S, 256, 256).astype(jnp.bfloat16)

    Wm = w_conv.astype(jnp.bfloat16)
    del b_conv

    y_pre, s1, s2 = pl.pallas_call(
        _msg_kernel,
        out_shape=(jax.ShapeDtypeStruct((N, O, L), jnp.bfloat16),
                   jax.ShapeDtypeStruct((N, O, 1), jnp.float32),
                   jax.ShapeDtypeStruct((N, O, 1), jnp.float32)),
        grid=(N // _NB,),
        in_specs=[pl.BlockSpec((_NB, C, TV), lambda i: (i, 0, 0)),
                  pl.BlockSpec((S, 256, 256), lambda i: (0, 0, 0)),
                  pl.BlockSpec((O, S * C), lambda i: (0, 0))],
        out_specs=(pl.BlockSpec((_NB, O, L), lambda i: (i, 0, 0)),
                   pl.BlockSpec((_NB, O, 1), lambda i: (i, 0, 0)),
                   pl.BlockSpec((_NB, O, 1), lambda i: (i, 0, 0))),
        scratch_shapes=[pltpu.VMEM((_NB * C, L), jnp.bfloat16),
                        pltpu.VMEM((S, _NB * C, L), jnp.bfloat16)],
        compiler_params=pltpu.CompilerParams(
            dimension_semantics=("parallel",),
            vmem_limit_bytes=64 * 1024 * 1024),
    )(xt, B, Wm)

    out = pl.pallas_call(
        _bn_kernel,
        out_shape=jax.ShapeDtypeStruct((N, O, TV), jnp.float32),
        grid=(N // _NB,),
        in_specs=[pl.BlockSpec((_NB, O, L), lambda i: (i, 0, 0)),
                  pl.BlockSpec((N, O, 1), lambda i: (0, 0, 0)),
                  pl.BlockSpec((N, O, 1), lambda i: (0, 0, 0)),
                  pl.BlockSpec((1, O), lambda i: (0, 0)),
                  pl.BlockSpec((1, O), lambda i: (0, 0))],
        out_specs=pl.BlockSpec((_NB, O, TV), lambda i: (i, 0, 0)),
        compiler_params=pltpu.CompilerParams(
            dimension_semantics=("parallel",)),
    )(y_pre, s1, s2, gamma.reshape(1, O), beta.reshape(1, O))

    return out.reshape(N, O, T, V)
